# Initial kernel scaffold; baseline (speedup 1.0000x reference)
import jax
import jax.numpy as jnp
from jax.experimental import pallas as pl


def kernel(z, codebook):
    raise NotImplementedError("write your pallas kernel here")


# speedup vs baseline: 1.0138x; 1.0138x over previous
import functools

import jax
import jax.numpy as jnp
from jax import lax
from jax.experimental import pallas as pl
from jax.experimental.pallas import tpu as pltpu
from jax.experimental.pallas import tpu_sc as plsc

_RT = 512
_KT = 2048


def _argmin_tile(zsq_ref, z_ref, cbt_ref, csq_ref, idx_ref, rmin_ref, ridx_ref):
    k = pl.program_id(1)
    nk = pl.num_programs(1)
    kt = cbt_ref.shape[1]
    ktot = kt * nk

    @pl.when(k == 0)
    def _init():
        rmin_ref[...] = jnp.full_like(rmin_ref, jnp.inf)
        ridx_ref[...] = jnp.zeros_like(ridx_ref)

    zc = lax.dot_general(
        z_ref[...].astype(jnp.bfloat16), cbt_ref[...].astype(jnp.bfloat16),
        dimension_numbers=(((1,), (0,)), ((), ())),
        preferred_element_type=jnp.float32)
    d = zsq_ref[...] + csq_ref[...] - 2.0 * zc
    tmin = jnp.min(d, axis=1, keepdims=True)
    cols = lax.broadcasted_iota(jnp.int32, d.shape, 1) + k * kt
    tidx = jnp.min(jnp.where(d == tmin, cols, jnp.int32(ktot)), axis=1,
                   keepdims=True)
    better = tmin < rmin_ref[...]
    tq = tmin.astype(jnp.bfloat16).astype(jnp.float32)
    rmin_ref[...] = jnp.where(better, tq, rmin_ref[...])
    ridx_ref[...] = jnp.where(better, tidx, ridx_ref[...])

    @pl.when(k == nk - 1)
    def _flush():
        idx_ref[...] = ridx_ref[...]


def _nearest_code_indices(zsq, z_flat, cbt, csq):
    n, d = z_flat.shape
    ktot = cbt.shape[1]
    grid = (n // _RT, ktot // _KT)
    return pl.pallas_call(
        _argmin_tile,
        grid=grid,
        in_specs=[
            pl.BlockSpec((_RT, 1), lambda i, k: (i, 0)),
            pl.BlockSpec((_RT, d), lambda i, k: (i, 0)),
            pl.BlockSpec((d, _KT), lambda i, k: (0, k)),
            pl.BlockSpec((1, _KT), lambda i, k: (0, k)),
        ],
        out_specs=pl.BlockSpec((_RT, 1), lambda i, k: (i, 0)),
        out_shape=jax.ShapeDtypeStruct((n, 1), jnp.int32),
        scratch_shapes=[
            pltpu.VMEM((_RT, 1), jnp.float32),
            pltpu.VMEM((_RT, 1), jnp.int32),
        ],
    )(zsq, z_flat, cbt, csq)


def _gather_codes(codebook, indices):
    n = indices.shape[0]
    d = codebook.shape[1]
    info = plsc.get_sparse_core_info()
    nw = info.num_cores * info.num_subcores
    b_per_w = n // nw
    mesh = plsc.VectorSubcoreMesh(core_axis_name="c", subcore_axis_name="s")

    @functools.partial(
        pl.kernel,
        mesh=mesh,
        out_type=jax.ShapeDtypeStruct((n, d), jnp.float32),
        scratch_types=[
            pltpu.VMEM((b_per_w,), jnp.int32),
            pltpu.VMEM((b_per_w, d), jnp.float32),
            pltpu.SemaphoreType.DMA,
        ],
        compiler_params=pltpu.CompilerParams(use_tc_tiling_on_sc=False),
    )
    def _sc_gather(table_hbm, idx_hbm, out_hbm, idx_v, rows_v, sem):
        wid = lax.axis_index("s") * info.num_cores + lax.axis_index("c")
        base = wid * b_per_w
        pltpu.sync_copy(idx_hbm.at[pl.ds(base, b_per_w)], idx_v)
        pltpu.async_copy(table_hbm.at[idx_v], rows_v, sem).wait()
        pltpu.sync_copy(rows_v, out_hbm.at[pl.ds(base, b_per_w)])

    return _sc_gather(codebook, indices)


def kernel(z, codebook):
    b, s, d = z.shape
    z_flat = z.reshape(b * s, d)
    zsq = (z_flat ** 2).sum(axis=1, keepdims=True)
    csq = (codebook ** 2).sum(axis=1)[None, :]
    cbt = codebook.T
    idx = _nearest_code_indices(zsq, z_flat, cbt, csq).reshape(b * s)
    z_q = _gather_codes(codebook, idx).reshape(b, s, d)
    z_q_st = z + lax.stop_gradient(z_q - z)
    return z_q_st, idx.reshape(b, s)


# speedup vs baseline: 1.2257x; 1.2091x over previous
import functools

import jax
import jax.numpy as jnp
from jax import lax
from jax.experimental import pallas as pl
from jax.experimental.pallas import tpu as pltpu
from jax.experimental.pallas import tpu_sc as plsc

_RT = 512
_KT = 2048


def _argmin_tile(zsq_ref, zb_ref, cbt_ref, csq_ref, cols_ref, idx_ref):
    ktot = cbt_ref.shape[1]
    zsq = zsq_ref[...]
    zb = zb_ref[...]
    rmin = jnp.full((zb.shape[0], 1), jnp.inf, jnp.float32)
    ridx = jnp.zeros((zb.shape[0], 1), jnp.float32)
    for k in range(ktot // _KT):
        zc = lax.dot_general(
            zb, cbt_ref[:, k * _KT:(k + 1) * _KT],
            dimension_numbers=(((1,), (0,)), ((), ())),
            preferred_element_type=jnp.float32)
        s = zsq + csq_ref[:, k * _KT:(k + 1) * _KT]
        d = s - zc
        tmin = jnp.min(d, axis=1, keepdims=True)
        cols = cols_ref[:, k * _KT:(k + 1) * _KT]
        tidx = jnp.min(jnp.where(d == tmin, cols, jnp.inf), axis=1,
                       keepdims=True)
        better = tmin < rmin
        rmin = jnp.where(better, tmin.astype(jnp.bfloat16).astype(jnp.float32),
                         rmin)
        ridx = jnp.where(better, tidx, ridx)
    idx_ref[...] = ridx.astype(jnp.int32)


def _nearest_code_indices(zsq, zb, cbt_b, csq):
    n, d = zb.shape
    ktot = cbt_b.shape[1]
    cols = lax.broadcasted_iota(jnp.float32, (1, ktot), 1)
    return pl.pallas_call(
        _argmin_tile,
        grid=(n // _RT,),
        in_specs=[
            pl.BlockSpec((_RT, 1), lambda i: (i, 0)),
            pl.BlockSpec((_RT, d), lambda i: (i, 0)),
            pl.BlockSpec((d, ktot), lambda i: (0, 0)),
            pl.BlockSpec((1, ktot), lambda i: (0, 0)),
            pl.BlockSpec((1, ktot), lambda i: (0, 0)),
        ],
        out_specs=pl.BlockSpec((_RT, 1), lambda i: (i, 0)),
        out_shape=jax.ShapeDtypeStruct((n, 1), jnp.int32),
    )(zsq, zb, cbt_b, csq, cols)


def _gather_codes(codebook, indices):
    n = indices.shape[0]
    d = codebook.shape[1]
    info = plsc.get_sparse_core_info()
    nw = info.num_cores * info.num_subcores
    b_per_w = n // nw
    mesh = plsc.VectorSubcoreMesh(core_axis_name="c", subcore_axis_name="s")

    @functools.partial(
        pl.kernel,
        mesh=mesh,
        out_type=jax.ShapeDtypeStruct((n, d), jnp.float32),
        scratch_types=[
            pltpu.VMEM((b_per_w,), jnp.int32),
            pltpu.VMEM((b_per_w, d), jnp.float32),
            pltpu.SemaphoreType.DMA,
        ],
        compiler_params=pltpu.CompilerParams(use_tc_tiling_on_sc=False),
    )
    def _sc_gather(table_hbm, idx_hbm, out_hbm, idx_v, rows_v, sem):
        wid = lax.axis_index("s") * info.num_cores + lax.axis_index("c")
        base = wid * b_per_w
        pltpu.sync_copy(idx_hbm.at[pl.ds(base, b_per_w)], idx_v)
        pltpu.async_copy(table_hbm.at[idx_v], rows_v, sem).wait()
        pltpu.sync_copy(rows_v, out_hbm.at[pl.ds(base, b_per_w)])

    return _sc_gather(codebook, indices)


def kernel(z, codebook):
    b, s, d = z.shape
    z_flat = z.reshape(b * s, d)
    zsq = (z_flat ** 2).sum(axis=1, keepdims=True)
    csq = (codebook ** 2).sum(axis=1)[None, :]
    zb = (2.0 * z_flat).astype(jnp.bfloat16)
    cbt_b = codebook.T.astype(jnp.bfloat16)
    idx = _nearest_code_indices(zsq, zb, cbt_b, csq).reshape(b * s)
    z_q = _gather_codes(codebook, idx).reshape(b, s, d)
    z_q_st = z + lax.stop_gradient(z_q - z)
    return z_q_st, idx.reshape(b, s)


# speedup vs baseline: 1.3805x; 1.1263x over previous
import functools

import jax
import jax.numpy as jnp
from jax import lax
from jax.experimental import pallas as pl
from jax.experimental.pallas import tpu as pltpu
from jax.experimental.pallas import tpu_sc as plsc

_RT = 512
_KT = 2048


def _argmin_tile(zsq_ref, zb_ref, cbt_ref, csq_ref, cols_ref, idx_ref):
    ktot = cbt_ref.shape[1]
    zsq = zsq_ref[...]
    zb = zb_ref[...]
    rmin = jnp.full((zb.shape[0], 1), jnp.inf, jnp.float32)
    ridx = jnp.zeros((zb.shape[0], 1), jnp.float32)
    lane = cols_ref[:, 0:128]
    for k in range(ktot // _KT):
        zc = lax.dot_general(
            zb, cbt_ref[:, k * _KT:(k + 1) * _KT],
            dimension_numbers=(((1,), (0,)), ((), ())),
            preferred_element_type=jnp.float32)
        mval = (zsq + csq_ref[:, k * _KT:k * _KT + 128]) - zc[:, 0:128]
        mjf = jnp.zeros_like(mval)
        for j in range(1, _KT // 128):
            lo = k * _KT + j * 128
            cur = (zsq + csq_ref[:, lo:lo + 128]) - zc[:, j * 128:(j + 1) * 128]
            lt = cur < mval
            mval = jnp.where(lt, cur, mval)
            mjf = jnp.where(lt, jnp.float32(j), mjf)
        tmin = jnp.min(mval, axis=1, keepdims=True)
        cand = jnp.where(mval == tmin, mjf * 128.0 + lane + jnp.float32(k * _KT),
                         jnp.inf)
        tidx = jnp.min(cand, axis=1, keepdims=True)
        better = tmin < rmin
        rmin = jnp.where(better, tmin.astype(jnp.bfloat16).astype(jnp.float32),
                         rmin)
        ridx = jnp.where(better, tidx, ridx)
    idx_ref[...] = ridx.astype(jnp.int32)


def _nearest_code_indices(zsq, zb, cbt_b, csq):
    n, d = zb.shape
    ktot = cbt_b.shape[1]
    cols = lax.broadcasted_iota(jnp.float32, (1, ktot), 1)
    return pl.pallas_call(
        _argmin_tile,
        grid=(n // _RT,),
        in_specs=[
            pl.BlockSpec((_RT, 1), lambda i: (i, 0)),
            pl.BlockSpec((_RT, d), lambda i: (i, 0)),
            pl.BlockSpec((d, ktot), lambda i: (0, 0)),
            pl.BlockSpec((1, ktot), lambda i: (0, 0)),
            pl.BlockSpec((1, ktot), lambda i: (0, 0)),
        ],
        out_specs=pl.BlockSpec((_RT, 1), lambda i: (i, 0)),
        out_shape=jax.ShapeDtypeStruct((n, 1), jnp.int32),
    )(zsq, zb, cbt_b, csq, cols)


def _gather_codes(codebook, indices):
    n = indices.shape[0]
    d = codebook.shape[1]
    info = plsc.get_sparse_core_info()
    nw = info.num_cores * info.num_subcores
    b_per_w = n // nw
    mesh = plsc.VectorSubcoreMesh(core_axis_name="c", subcore_axis_name="s")

    @functools.partial(
        pl.kernel,
        mesh=mesh,
        out_type=jax.ShapeDtypeStruct((n, d), jnp.float32),
        scratch_types=[
            pltpu.VMEM((b_per_w,), jnp.int32),
            pltpu.VMEM((b_per_w, d), jnp.float32),
            pltpu.SemaphoreType.DMA,
        ],
        compiler_params=pltpu.CompilerParams(use_tc_tiling_on_sc=False),
    )
    def _sc_gather(table_hbm, idx_hbm, out_hbm, idx_v, rows_v, sem):
        wid = lax.axis_index("s") * info.num_cores + lax.axis_index("c")
        base = wid * b_per_w
        pltpu.sync_copy(idx_hbm.at[pl.ds(base, b_per_w)], idx_v)
        pltpu.async_copy(table_hbm.at[idx_v], rows_v, sem).wait()
        pltpu.sync_copy(rows_v, out_hbm.at[pl.ds(base, b_per_w)])

    return _sc_gather(codebook, indices)


def kernel(z, codebook):
    b, s, d = z.shape
    z_flat = z.reshape(b * s, d)
    zsq = (z_flat ** 2).sum(axis=1, keepdims=True)
    csq = (codebook ** 2).sum(axis=1)[None, :]
    zb = (2.0 * z_flat).astype(jnp.bfloat16)
    cbt_b = codebook.T.astype(jnp.bfloat16)
    idx = _nearest_code_indices(zsq, zb, cbt_b, csq).reshape(b * s)
    z_q = _gather_codes(codebook, idx).reshape(b, s, d)
    z_q_st = z + lax.stop_gradient(z_q - z)
    return z_q_st, idx.reshape(b, s)


# speedup vs baseline: 1.4183x; 1.0274x over previous
import functools

import jax
import jax.numpy as jnp
from jax import lax
from jax.experimental import pallas as pl
from jax.experimental.pallas import tpu as pltpu
from jax.experimental.pallas import tpu_sc as plsc

_RT = 1024
_KT = 2048


def _argmin_tile(zsq_ref, zb_ref, cbt_ref, csq_ref, cols_ref, idx_ref):
    ktot = cbt_ref.shape[1]
    zsq = zsq_ref[...]
    zb = zb_ref[...]
    rmin = jnp.full((zb.shape[0], 1), jnp.inf, jnp.float32)
    ridx = jnp.zeros((zb.shape[0], 1), jnp.float32)
    lane = cols_ref[:, 0:128]
    for k in range(ktot // _KT):
        zc = lax.dot_general(
            zb, cbt_ref[:, k * _KT:(k + 1) * _KT],
            dimension_numbers=(((1,), (0,)), ((), ())),
            preferred_element_type=jnp.float32)
        mval = (zsq + csq_ref[:, k * _KT:k * _KT + 128]) - zc[:, 0:128]
        mjf = jnp.zeros_like(mval)
        for j in range(1, _KT // 128):
            lo = k * _KT + j * 128
            cur = (zsq + csq_ref[:, lo:lo + 128]) - zc[:, j * 128:(j + 1) * 128]
            lt = cur < mval
            mval = jnp.where(lt, cur, mval)
            mjf = jnp.where(lt, jnp.float32(j), mjf)
        tmin = jnp.min(mval, axis=1, keepdims=True)
        cand = jnp.where(mval == tmin, mjf * 128.0 + lane + jnp.float32(k * _KT),
                         jnp.inf)
        tidx = jnp.min(cand, axis=1, keepdims=True)
        better = tmin < rmin
        rmin = jnp.where(better, tmin.astype(jnp.bfloat16).astype(jnp.float32),
                         rmin)
        ridx = jnp.where(better, tidx, ridx)
    idx_ref[...] = ridx.astype(jnp.int32)


def _nearest_code_indices(zsq, zb, cbt_b, csq):
    n, d = zb.shape
    ktot = cbt_b.shape[1]
    cols = lax.broadcasted_iota(jnp.float32, (1, ktot), 1)
    return pl.pallas_call(
        _argmin_tile,
        grid=(n // _RT,),
        in_specs=[
            pl.BlockSpec((_RT, 1), lambda i: (i, 0)),
            pl.BlockSpec((_RT, d), lambda i: (i, 0)),
            pl.BlockSpec((d, ktot), lambda i: (0, 0)),
            pl.BlockSpec((1, ktot), lambda i: (0, 0)),
            pl.BlockSpec((1, ktot), lambda i: (0, 0)),
        ],
        out_specs=pl.BlockSpec((_RT, 1), lambda i: (i, 0)),
        out_shape=jax.ShapeDtypeStruct((n, 1), jnp.int32),
    )(zsq, zb, cbt_b, csq, cols)


def _gather_codes(codebook, indices):
    n = indices.shape[0]
    d = codebook.shape[1]
    info = plsc.get_sparse_core_info()
    nw = info.num_cores * info.num_subcores
    b_per_w = n // nw
    mesh = plsc.VectorSubcoreMesh(core_axis_name="c", subcore_axis_name="s")

    @functools.partial(
        pl.kernel,
        mesh=mesh,
        out_type=jax.ShapeDtypeStruct((n, d), jnp.float32),
        scratch_types=[
            pltpu.VMEM((b_per_w,), jnp.int32),
            pltpu.VMEM((b_per_w, d), jnp.float32),
            pltpu.SemaphoreType.DMA,
        ],
        compiler_params=pltpu.CompilerParams(use_tc_tiling_on_sc=False),
    )
    def _sc_gather(table_hbm, idx_hbm, out_hbm, idx_v, rows_v, sem):
        wid = lax.axis_index("s") * info.num_cores + lax.axis_index("c")
        base = wid * b_per_w
        pltpu.sync_copy(idx_hbm.at[pl.ds(base, b_per_w)], idx_v)
        pltpu.async_copy(table_hbm.at[idx_v], rows_v, sem).wait()
        pltpu.sync_copy(rows_v, out_hbm.at[pl.ds(base, b_per_w)])

    return _sc_gather(codebook, indices)


def kernel(z, codebook):
    b, s, d = z.shape
    z_flat = z.reshape(b * s, d)
    zsq = (z_flat ** 2).sum(axis=1, keepdims=True)
    csq = (codebook ** 2).sum(axis=1)[None, :]
    zb = (2.0 * z_flat).astype(jnp.bfloat16)
    cbt_b = codebook.T.astype(jnp.bfloat16)
    idx = _nearest_code_indices(zsq, zb, cbt_b, csq).reshape(b * s)
    z_q = _gather_codes(codebook, idx).reshape(b, s, d)
    z_q_st = z + lax.stop_gradient(z_q - z)
    return z_q_st, idx.reshape(b, s)


# speedup vs baseline: 1.4603x; 1.0296x over previous
import functools

import jax
import jax.numpy as jnp
from jax import lax
from jax.experimental import pallas as pl
from jax.experimental.pallas import tpu as pltpu
from jax.experimental.pallas import tpu_sc as plsc

_RT = 2048
_KT = 2048


def _argmin_tile(zsq_ref, zb_ref, cbt_ref, csq_ref, cols_ref, idx_ref):
    ktot = cbt_ref.shape[1]
    zsq = zsq_ref[...]
    zb = zb_ref[...]
    rmin = jnp.full((zb.shape[0], 1), jnp.inf, jnp.float32)
    ridx = jnp.zeros((zb.shape[0], 1), jnp.float32)
    lane = cols_ref[:, 0:128]
    for k in range(ktot // _KT):
        zc = lax.dot_general(
            zb, cbt_ref[:, k * _KT:(k + 1) * _KT],
            dimension_numbers=(((1,), (0,)), ((), ())),
            preferred_element_type=jnp.float32)
        mval = (zsq + csq_ref[:, k * _KT:k * _KT + 128]) - zc[:, 0:128]
        mjf = jnp.zeros_like(mval)
        for j in range(1, _KT // 128):
            lo = k * _KT + j * 128
            cur = (zsq + csq_ref[:, lo:lo + 128]) - zc[:, j * 128:(j + 1) * 128]
            lt = cur < mval
            mval = jnp.where(lt, cur, mval)
            mjf = jnp.where(lt, jnp.float32(j), mjf)
        tmin = jnp.min(mval, axis=1, keepdims=True)
        cand = jnp.where(mval == tmin, mjf * 128.0 + lane + jnp.float32(k * _KT),
                         jnp.inf)
        tidx = jnp.min(cand, axis=1, keepdims=True)
        better = tmin < rmin
        rmin = jnp.where(better, tmin.astype(jnp.bfloat16).astype(jnp.float32),
                         rmin)
        ridx = jnp.where(better, tidx, ridx)
    idx_ref[...] = ridx.astype(jnp.int32)


def _nearest_code_indices(zsq, zb, cbt_b, csq):
    n, d = zb.shape
    ktot = cbt_b.shape[1]
    cols = lax.broadcasted_iota(jnp.float32, (1, ktot), 1)
    return pl.pallas_call(
        _argmin_tile,
        grid=(n // _RT,),
        in_specs=[
            pl.BlockSpec((_RT, 1), lambda i: (i, 0)),
            pl.BlockSpec((_RT, d), lambda i: (i, 0)),
            pl.BlockSpec((d, ktot), lambda i: (0, 0)),
            pl.BlockSpec((1, ktot), lambda i: (0, 0)),
            pl.BlockSpec((1, ktot), lambda i: (0, 0)),
        ],
        out_specs=pl.BlockSpec((_RT, 1), lambda i: (i, 0)),
        out_shape=jax.ShapeDtypeStruct((n, 1), jnp.int32),
    )(zsq, zb, cbt_b, csq, cols)


def _gather_codes(codebook, indices):
    n = indices.shape[0]
    d = codebook.shape[1]
    info = plsc.get_sparse_core_info()
    nw = info.num_cores * info.num_subcores
    b_per_w = n // nw
    mesh = plsc.VectorSubcoreMesh(core_axis_name="c", subcore_axis_name="s")

    @functools.partial(
        pl.kernel,
        mesh=mesh,
        out_type=jax.ShapeDtypeStruct((n, d), jnp.float32),
        scratch_types=[
            pltpu.VMEM((b_per_w,), jnp.int32),
            pltpu.VMEM((b_per_w, d), jnp.float32),
            pltpu.SemaphoreType.DMA,
        ],
        compiler_params=pltpu.CompilerParams(use_tc_tiling_on_sc=False),
    )
    def _sc_gather(table_hbm, idx_hbm, out_hbm, idx_v, rows_v, sem):
        wid = lax.axis_index("s") * info.num_cores + lax.axis_index("c")
        base = wid * b_per_w
        pltpu.sync_copy(idx_hbm.at[pl.ds(base, b_per_w)], idx_v)
        pltpu.async_copy(table_hbm.at[idx_v], rows_v, sem).wait()
        pltpu.sync_copy(rows_v, out_hbm.at[pl.ds(base, b_per_w)])

    return _sc_gather(codebook, indices)


def kernel(z, codebook):
    b, s, d = z.shape
    z_flat = z.reshape(b * s, d)
    zsq = (z_flat ** 2).sum(axis=1, keepdims=True)
    csq = (codebook ** 2).sum(axis=1)[None, :]
    zb = (2.0 * z_flat).astype(jnp.bfloat16)
    cbt_b = codebook.T.astype(jnp.bfloat16)
    idx = _nearest_code_indices(zsq, zb, cbt_b, csq).reshape(b * s)
    z_q = _gather_codes(codebook, idx).reshape(b, s, d)
    z_q_st = z + lax.stop_gradient(z_q - z)
    return z_q_st, idx.reshape(b, s)
